# baseline (speedup 1.0000x reference)
import functools
import math

import jax
import jax.numpy as jnp
from jax import lax
from jax.experimental import pallas as pl
from jax.experimental.pallas import tpu as pltpu
from jax.experimental.pallas import tpu_sc as plsc


def _sc_gather(ntok, din, nw, nch, ch):
    per_w = nch * ch
    mesh = plsc.VectorSubcoreMesh(core_axis_name="c", subcore_axis_name="s")

    @functools.partial(
        pl.kernel,
        mesh=mesh,
        out_type=jax.ShapeDtypeStruct((ntok, din), jnp.float32),
        scratch_types=[
            pltpu.VMEM((nch, ch), jnp.int32),
            pltpu.VMEM((ch, din), jnp.float32),
            pltpu.VMEM((ch, din), jnp.float32),
            pltpu.VMEM((ch, din), jnp.float32),
            pltpu.SemaphoreType.DMA,
            pltpu.SemaphoreType.DMA,
            pltpu.SemaphoreType.DMA,
            pltpu.SemaphoreType.DMA,
            pltpu.SemaphoreType.DMA,
            pltpu.SemaphoreType.DMA,
        ],
    )
    def gather(idx_hbm, table_hbm, out_hbm, idx_v,
               b0, b1, b2, gs0, gs1, gs2, ws0, ws1, ws2):
        info = plsc.get_sparse_core_info()
        wid = lax.axis_index("s") * info.num_cores + lax.axis_index("c")
        base = wid * per_w
        pltpu.sync_copy(idx_hbm.at[wid], idx_v)
        bufs = (b0, b1, b2)
        gsems = (gs0, gs1, gs2)
        wsems = (ws0, ws1, ws2)
        gcp, wcp = {}, {}
        for p in range(min(2, nch)):
            gcp[p] = pltpu.async_copy(table_hbm.at[idx_v.at[p]], bufs[p], gsems[p])
        for c in range(nch):
            gcp[c].wait()
            wcp[c] = pltpu.async_copy(
                bufs[c % 3], out_hbm.at[pl.ds(base + c * ch, ch)], wsems[c % 3])
            nxt = c + 2
            if nxt < nch:
                if c >= 1:
                    wcp[c - 1].wait()
                gcp[nxt] = pltpu.async_copy(
                    table_hbm.at[idx_v.at[nxt]], bufs[nxt % 3], gsems[nxt % 3])
        for c in range(max(0, nch - 3), nch):
            wcp[c].wait()

    return gather


def _tc_project(bsz, seq, din, dout, rb, scale):
    sb = rb // bsz

    def body(x_ref, w_ref, o_ref):
        w = w_ref[...].astype(jnp.bfloat16)
        y = lax.dot_general(
            x_ref[...].astype(jnp.bfloat16), w,
            (((1,), (1,)), ((), ())),
            preferred_element_type=jnp.float32) * scale
        o_ref[...] = y.reshape(sb, bsz, dout)

    return pl.pallas_call(
        body,
        grid=(bsz * seq // rb,),
        in_specs=[
            pl.BlockSpec((rb, din), lambda i: (i, 0)),
            pl.BlockSpec((dout, din), lambda i: (0, 0)),
        ],
        out_specs=pl.BlockSpec((sb, bsz, dout), lambda i: (i, 0, 0)),
        out_shape=jax.ShapeDtypeStruct((seq, bsz, dout), jnp.float32),
    )


def kernel(input, embed_weight, proj_weight):
    bsz, seq = input.shape
    _, din = embed_weight.shape
    dout = proj_weight.shape[0]
    scale = math.sqrt(float(dout))
    ntok = bsz * seq

    nw = 32
    ch = 32
    per_w = ntok // nw
    nch = per_w // ch

    idx3 = jnp.transpose(input).reshape(nw, nch, ch)
    gathered = _sc_gather(ntok, din, nw, nch, ch)(idx3, embed_weight)
    return _tc_project(bsz, seq, din, dout, 1024, scale)(gathered, proj_weight)
